# baseline (speedup 1.0000x reference)
import functools

import jax
import jax.numpy as jnp
from jax import lax
from jax.experimental import pallas as pl
from jax.experimental.pallas import tpu as pltpu
from jax.experimental.pallas import tpu_sc as plsc

_HID = 64
_SCALE = 8.0
_NC = 2
_NS = 16
_NW = _NC * _NS
_L16 = 16
_CHUNK = 128


def _sc_relayout(vocab: int, hid: int):
    n_vc = vocab // _CHUNK
    n_q = n_vc // 2
    steps = 124
    out_rows = (vocab // 2 + 63) // 64 * 64
    tail_v = n_vc * _CHUNK
    tail_n = vocab - tail_v
    mesh = plsc.VectorSubcoreMesh(core_axis_name="c", subcore_axis_name="s")

    @functools.partial(
        pl.kernel,
        mesh=mesh,
        out_type=jax.ShapeDtypeStruct((out_rows, _CHUNK), jnp.float32),
        scratch_types=[
            pltpu.VMEM((2, hid, 2 * _CHUNK), jnp.float32),
            pltpu.VMEM((2, 2 * hid, _CHUNK), jnp.float32),
            pltpu.SemaphoreType.DMA,
            pltpu.SemaphoreType.DMA,
            pltpu.SemaphoreType.DMA,
            pltpu.SemaphoreType.DMA,
        ],
        compiler_params=pltpu.CompilerParams(needs_layout_passes=False),
    )
    def k(tab_hbm, tail_hbm, out_hbm, sbuf, obuf, g0, g1, s0, s1):
        wid = lax.axis_index("s") * _NC + lax.axis_index("c")
        gsem = (g0, g1)
        ssem = (s0, s1)

        def q_of(s):
            return wid + _NW * s

        def gstart(s, b):
            pltpu.async_copy(
                tab_hbm.at[:, pl.ds(q_of(s) * 2 * _CHUNK, 2 * _CHUNK)],
                sbuf.at[b], gsem[b])

        def gwait(s, b):
            pltpu.make_async_copy(
                tab_hbm.at[:, pl.ds(q_of(s) * 2 * _CHUNK, 2 * _CHUNK)],
                sbuf.at[b], gsem[b]).wait()

        def sstart(s, b):
            pltpu.async_copy(obuf.at[b],
                             out_hbm.at[pl.ds(q_of(s) * 2 * 64, 2 * 64)],
                             ssem[b])

        def swait(s, b):
            pltpu.make_async_copy(
                obuf.at[b], out_hbm.at[pl.ds(q_of(s) * 2 * 64, 2 * 64)],
                ssem[b]).wait()

        iota = lax.iota(jnp.int32, _L16)

        def transpose(b):
            for j in range(2):
                for vg in range(_CHUNK // _L16):
                    vlv = vg * _L16 + iota
                    colv = j * _CHUNK + vlv
                    rowb = j * 64 + lax.shift_right_logical(vlv, 1)
                    colb = (vlv & 1) * hid

                    @plsc.parallel_loop(0, 64, unroll=8)
                    def _d(d, colv=colv, rowb=rowb, colb=colb):
                        hvec = (iota + d) & 63
                        vals = plsc.load_gather(sbuf.at[b], [hvec, colv])
                        plsc.store_scatter(obuf.at[b],
                                           [rowb, colb + hvec], vals)

        gstart(0, 0)

        def body(g, carry):
            for b in range(2):
                s = g + b

                @pl.when(q_of(s) < n_q)
                def _step(s=s, b=b):
                    gwait(s, b)

                    @pl.when(q_of(s + 1) < n_q)
                    def _pref(s=s, b=b):
                        gstart(s + 1, 1 - b)

                    @pl.when(s >= 2)
                    def _drain(s=s, b=b):
                        swait(s - 2, b)

                    transpose(b)
                    sstart(s, b)
            return carry

        lax.fori_loop(0, steps // 2, lambda i, c: body(i * 2, c), 0)

        n_valid = (n_q - wid + _NW - 1) // _NW
        for p in range(2):
            s_p = n_valid - 1 - ((n_valid - 1 - p) & 1)
            swait(s_p, p)

        @pl.when(wid == _NW - 1)
        def _tail():
            pltpu.sync_copy(tail_hbm, obuf.at[0, pl.ds(0, tail_n // 2)])
            pltpu.sync_copy(obuf.at[0, pl.ds(0, tail_n // 2)],
                            out_hbm.at[pl.ds(tail_v // 2, tail_n // 2)])

    return k


def _sc_embed(n_l: int, n_bt: int, vocab_padded: int):
    chunks = n_l * n_bt
    steps = chunks // _NW
    per_w = steps * _CHUNK
    mesh = plsc.VectorSubcoreMesh(core_axis_name="c", subcore_axis_name="s")

    @functools.partial(
        pl.kernel,
        mesh=mesh,
        out_type=jax.ShapeDtypeStruct((n_l, 8, n_bt, 8, _CHUNK), jnp.float32),
        scratch_types=[
            pltpu.VMEM((per_w,), jnp.int32),
            pltpu.VMEM((4, _CHUNK, _HID), jnp.float32),
            pltpu.VMEM((2, 8, 8, _CHUNK), jnp.float32),
            pltpu.SemaphoreType.DMA,
            pltpu.SemaphoreType.DMA,
            pltpu.SemaphoreType.DMA,
            pltpu.SemaphoreType.DMA,
            pltpu.SemaphoreType.DMA,
            pltpu.SemaphoreType.DMA,
        ],
        compiler_params=pltpu.CompilerParams(
            needs_layout_passes=False, use_tc_tiling_on_sc=False),
    )
    def k(ids_hbm, tab_hbm, out_hbm, idx_v, buf_v, obuf_v,
          g0, g1, g2, g3, s0, s1):
        wid = lax.axis_index("s") * _NC + lax.axis_index("c")
        base = wid * per_w
        gsem = (g0, g1, g2, g3)
        ssem = (s0, s1)

        pltpu.sync_copy(ids_hbm.at[pl.ds(base, per_w)], idx_v)

        def gstart(s, b):
            pltpu.async_copy(
                tab_hbm.at[idx_v.at[pl.ds(s * _CHUNK, _CHUNK)]],
                buf_v.at[b], gsem[b])

        def gwait(s, b):
            pltpu.make_async_copy(
                tab_hbm.at[idx_v.at[pl.ds(s * _CHUNK, _CHUNK)]],
                buf_v.at[b], gsem[b]).wait()

        def out_slices(s):
            q = wid * steps + s
            return lax.shift_right_logical(q, 7), q & (n_bt - 1)

        def sstart(s, b):
            l, bc = out_slices(s)
            pltpu.async_copy(obuf_v.at[b], out_hbm.at[l, :, bc], ssem[b])

        def swait(s, b):
            l, bc = out_slices(s)
            pltpu.make_async_copy(obuf_v.at[b], out_hbm.at[l, :, bc],
                                  ssem[b]).wait()

        def transpose_scale(s, b):
            ob = b % 2
            src = buf_v.at[b]
            iota = lax.iota(jnp.int32, _L16)
            for bg in range(8):
                rowv = iota + bg * _L16

                @plsc.parallel_loop(0, 64, unroll=8)
                def _d(d, rowv=rowv):
                    hvec = (iota + d) & 63
                    vals = plsc.load_gather(src, [rowv, hvec])
                    plsc.store_scatter(
                        obuf_v.at[ob],
                        [lax.shift_right_logical(hvec, 3), hvec & 7, rowv],
                        vals * _SCALE)

        for b0 in range(3):
            gstart(b0, b0)

        def body(g, carry):
            for b in range(4):
                s = g + b
                gwait(s, b)

                @pl.when(s + 3 < steps)
                def _pref(s=s, b=b):
                    gstart(s + 3, (b + 3) % 4)

                @pl.when(s >= 2)
                def _drain(s=s, b=b):
                    swait(s - 2, b % 2)

                transpose_scale(s, b)
                sstart(s, b % 2)
            return carry

        lax.fori_loop(0, steps // 4, lambda i, c: body(i * 4, c), 0)
        swait(steps - 2, 0)
        swait(steps - 1, 1)

    return k


def kernel(input_ids, embed_table):
    b, l = input_ids.shape
    vocab, hid = embed_table.shape
    total = b * l
    n_bt = b // _CHUNK
    ids_flat = input_ids.T.reshape(total).astype(jnp.int32)
    n_full = (vocab // _CHUNK) * _CHUNK
    tail_packed = embed_table[n_full:].reshape(-1, 2 * hid)
    tab2 = _sc_relayout(vocab, hid)(embed_table.T, tail_packed)
    vocab_padded = tab2.shape[0] * 2
    tab_rows = tab2.reshape(vocab_padded, hid)
    out5 = _sc_embed(l, n_bt, vocab_padded)(ids_flat, tab_rows)
    return out5.transpose(2, 4, 0, 1, 3).reshape(b, l, hid)


# speedup vs baseline: 1.0109x; 1.0109x over previous
import functools

import jax
import jax.numpy as jnp
from jax import lax
from jax.experimental import pallas as pl
from jax.experimental.pallas import tpu as pltpu
from jax.experimental.pallas import tpu_sc as plsc

_HID = 64
_SCALE = 8.0
_NC = 2
_NS = 16
_NW = _NC * _NS
_L16 = 16
_CHUNK = 128


def _sc_relayout(vocab: int, hid: int):
    n_vc = vocab // _CHUNK
    n_q = n_vc // 2
    steps = 124
    out_rows = (vocab // 2 + 63) // 64 * 64
    tail_v = n_vc * _CHUNK
    tail_n = vocab - tail_v
    mesh = plsc.VectorSubcoreMesh(core_axis_name="c", subcore_axis_name="s")

    @functools.partial(
        pl.kernel,
        mesh=mesh,
        out_type=jax.ShapeDtypeStruct((out_rows, _CHUNK), jnp.float32),
        scratch_types=[
            pltpu.VMEM((4, hid, 2 * _CHUNK), jnp.float32),
            pltpu.VMEM((2, 2 * hid, _CHUNK), jnp.float32),
            pltpu.SemaphoreType.DMA,
            pltpu.SemaphoreType.DMA,
            pltpu.SemaphoreType.DMA,
            pltpu.SemaphoreType.DMA,
            pltpu.SemaphoreType.DMA,
            pltpu.SemaphoreType.DMA,
        ],
        compiler_params=pltpu.CompilerParams(needs_layout_passes=False),
    )
    def k(tab_hbm, tail_hbm, out_hbm, sbuf, obuf, g0, g1, g2, g3, s0, s1):
        wid = lax.axis_index("s") * _NC + lax.axis_index("c")
        gsem = (g0, g1, g2, g3)
        ssem = (s0, s1)

        def q_of(s):
            return wid + _NW * s

        def gstart(s, b):
            pltpu.async_copy(
                tab_hbm.at[:, pl.ds(q_of(s) * 2 * _CHUNK, 2 * _CHUNK)],
                sbuf.at[b], gsem[b])

        def gwait(s, b):
            pltpu.make_async_copy(
                tab_hbm.at[:, pl.ds(q_of(s) * 2 * _CHUNK, 2 * _CHUNK)],
                sbuf.at[b], gsem[b]).wait()

        def sstart(s, b):
            pltpu.async_copy(obuf.at[b % 2],
                             out_hbm.at[pl.ds(q_of(s) * 2 * 64, 2 * 64)],
                             ssem[b % 2])

        def swait(s, b):
            pltpu.make_async_copy(
                obuf.at[b % 2], out_hbm.at[pl.ds(q_of(s) * 2 * 64, 2 * 64)],
                ssem[b % 2]).wait()

        iota = lax.iota(jnp.int32, _L16)

        def transpose(b, ob):
            for j in range(2):
                for vg in range(_CHUNK // _L16):
                    vlv = vg * _L16 + iota
                    colv = j * _CHUNK + vlv
                    rowb = j * 64 + lax.shift_right_logical(vlv, 1)
                    colb = (vlv & 1) * hid

                    @plsc.parallel_loop(0, 64, unroll=8)
                    def _d(d, colv=colv, rowb=rowb, colb=colb):
                        hvec = (iota + d) & 63
                        vals = plsc.load_gather(sbuf.at[b], [hvec, colv])
                        plsc.store_scatter(obuf.at[ob],
                                           [rowb, colb + hvec], vals)

        for b0 in range(3):
            gstart(b0, b0)

        def body(g, carry):
            for b in range(4):
                s = g + b

                @pl.when(q_of(s) < n_q)
                def _step(s=s, b=b):
                    gwait(s, b)

                    @pl.when(q_of(s + 3) < n_q)
                    def _pref(s=s, b=b):
                        gstart(s + 3, (b + 3) % 4)

                    @pl.when(s >= 2)
                    def _drain(s=s, b=b):
                        swait(s - 2, b)

                    transpose(b, b % 2)
                    sstart(s, b)
            return carry

        lax.fori_loop(0, steps // 4, lambda i, c: body(i * 4, c), 0)

        n_valid = (n_q - wid + _NW - 1) // _NW
        for p in range(2):
            s_p = n_valid - 1 - ((n_valid - 1 - p) & 1)
            swait(s_p, p)

        @pl.when(wid == _NW - 1)
        def _tail():
            pltpu.sync_copy(tail_hbm, obuf.at[0, pl.ds(0, tail_n // 2)])
            pltpu.sync_copy(obuf.at[0, pl.ds(0, tail_n // 2)],
                            out_hbm.at[pl.ds(tail_v // 2, tail_n // 2)])

    return k


def _sc_embed(n_l: int, n_bt: int, vocab_padded: int):
    chunks = n_l * n_bt
    steps = chunks // _NW
    per_w = steps * _CHUNK
    mesh = plsc.VectorSubcoreMesh(core_axis_name="c", subcore_axis_name="s")

    @functools.partial(
        pl.kernel,
        mesh=mesh,
        out_type=jax.ShapeDtypeStruct((n_l, 8, n_bt, 8, _CHUNK), jnp.float32),
        scratch_types=[
            pltpu.VMEM((per_w,), jnp.int32),
            pltpu.VMEM((4, _CHUNK, _HID), jnp.float32),
            pltpu.VMEM((2, 8, 8, _CHUNK), jnp.float32),
            pltpu.SemaphoreType.DMA,
            pltpu.SemaphoreType.DMA,
            pltpu.SemaphoreType.DMA,
            pltpu.SemaphoreType.DMA,
            pltpu.SemaphoreType.DMA,
            pltpu.SemaphoreType.DMA,
        ],
        compiler_params=pltpu.CompilerParams(
            needs_layout_passes=False, use_tc_tiling_on_sc=False),
    )
    def k(ids_hbm, tab_hbm, out_hbm, idx_v, buf_v, obuf_v,
          g0, g1, g2, g3, s0, s1):
        wid = lax.axis_index("s") * _NC + lax.axis_index("c")
        base = wid * per_w
        gsem = (g0, g1, g2, g3)
        ssem = (s0, s1)

        pltpu.sync_copy(ids_hbm.at[pl.ds(base, per_w)], idx_v)

        def gstart(s, b):
            pltpu.async_copy(
                tab_hbm.at[idx_v.at[pl.ds(s * _CHUNK, _CHUNK)]],
                buf_v.at[b], gsem[b])

        def gwait(s, b):
            pltpu.make_async_copy(
                tab_hbm.at[idx_v.at[pl.ds(s * _CHUNK, _CHUNK)]],
                buf_v.at[b], gsem[b]).wait()

        def out_slices(s):
            q = wid * steps + s
            return lax.shift_right_logical(q, 7), q & (n_bt - 1)

        def sstart(s, b):
            l, bc = out_slices(s)
            pltpu.async_copy(obuf_v.at[b], out_hbm.at[l, :, bc], ssem[b])

        def swait(s, b):
            l, bc = out_slices(s)
            pltpu.make_async_copy(obuf_v.at[b], out_hbm.at[l, :, bc],
                                  ssem[b]).wait()

        def transpose_scale(s, b):
            ob = b % 2
            src = buf_v.at[b]
            iota = lax.iota(jnp.int32, _L16)
            for bg in range(8):
                rowv = iota + bg * _L16

                @plsc.parallel_loop(0, 64, unroll=8)
                def _d(d, rowv=rowv):
                    hvec = (iota + d) & 63
                    vals = plsc.load_gather(src, [rowv, hvec])
                    plsc.store_scatter(
                        obuf_v.at[ob],
                        [lax.shift_right_logical(hvec, 3), hvec & 7, rowv],
                        vals * _SCALE)

        for b0 in range(3):
            gstart(b0, b0)

        def body(g, carry):
            for b in range(4):
                s = g + b
                gwait(s, b)

                @pl.when(s + 3 < steps)
                def _pref(s=s, b=b):
                    gstart(s + 3, (b + 3) % 4)

                @pl.when(s >= 2)
                def _drain(s=s, b=b):
                    swait(s - 2, b % 2)

                transpose_scale(s, b)
                sstart(s, b % 2)
            return carry

        lax.fori_loop(0, steps // 4, lambda i, c: body(i * 4, c), 0)
        swait(steps - 2, 0)
        swait(steps - 1, 1)

    return k


def kernel(input_ids, embed_table):
    b, l = input_ids.shape
    vocab, hid = embed_table.shape
    total = b * l
    n_bt = b // _CHUNK
    ids_flat = input_ids.T.reshape(total).astype(jnp.int32)
    n_full = (vocab // _CHUNK) * _CHUNK
    tail_packed = embed_table[n_full:].reshape(-1, 2 * hid)
    tab2 = _sc_relayout(vocab, hid)(embed_table.T, tail_packed)
    vocab_padded = tab2.shape[0] * 2
    tab_rows = tab2.reshape(vocab_padded, hid)
    out5 = _sc_embed(l, n_bt, vocab_padded)(ids_flat, tab_rows)
    return out5.transpose(2, 4, 0, 1, 3).reshape(b, l, hid)
